# baseline (speedup 1.0000x reference)
import functools

import jax
import jax.numpy as jnp
from jax import lax
from jax.experimental import pallas as pl
from jax.experimental.pallas import tpu as pltpu
from jax.experimental.pallas import tpu_sc as plsc

_N = 10000
_D = 256
_H = 256
_NPAD = 10240
_R = 512
_NSC = 2
_NTS = 16
_K = 128
_RSLICE = _NPAD // _NTS
_CB = 32
_GB = 8


def _tc_proj(x_pad, w_all, a_all):
    def body(x_ref, w_ref, a_ref, h_ref, s_ref):
        cid = pl.program_id(2)
        xb = x_ref[...]
        hb = jnp.dot(xb, w_ref[0, 0], preferred_element_type=jnp.float32)
        h_ref[0, 0] = hb
        sb = lax.dot_general(a_ref[0, 0], hb, (((1,), (1,)), ((), ())),
                             preferred_element_type=jnp.float32)

        @pl.when(cid == 0)
        def _():
            s_ref[0] = sb

        @pl.when(cid == 1)
        def _():
            s_ref[0] = s_ref[0] + sb

    nrb = _NPAD // _R
    return pl.pallas_call(
        body,
        grid=(3, nrb, 2),
        in_specs=[
            pl.BlockSpec((_R, _D), lambda h, r, c: (r, 0)),
            pl.BlockSpec((1, 1, _D, 128), lambda h, r, c: (h, c, 0, 0)),
            pl.BlockSpec((1, 1, 2, 128), lambda h, r, c: (h, c, 0, 0)),
        ],
        out_specs=[
            pl.BlockSpec((1, 1, _R, 128), lambda h, r, c: (h, c, r, 0)),
            pl.BlockSpec((1, 2, _R), lambda h, r, c: (h, 0, r)),
        ],
        out_shape=[
            jax.ShapeDtypeStruct((3, 2, _NPAD, 128), jnp.float32),
            jax.ShapeDtypeStruct((3, 2, _NPAD), jnp.float32),
        ],
    )(x_pad, w_all, a_all)


def _lrelu_exp(z):
    return jnp.exp(jnp.where(z >= 0, z, 0.2 * z))


def _bcast16(j):
    return jnp.zeros((16,), jnp.int32) + j


_MESH = plsc.VectorSubcoreMesh(core_axis_name="c", subcore_axis_name="s",
                               num_cores=_NSC, num_subcores=_NTS)


def _make_alpha_kernel(epad):
    nblk_den = epad // (_NTS * _K)
    nblk_al = epad // (_NSC * _NTS * _K)

    @functools.partial(
        pl.kernel,
        out_type=jax.ShapeDtypeStruct((3 * epad,), jnp.float32),
        mesh=_MESH,
        scratch_types=[
            pltpu.VMEM((_NPAD,), jnp.float32),
            pltpu.VMEM((_NPAD,), jnp.float32),
            pltpu.VMEM((_NPAD,), jnp.float32),
            pltpu.VMEM((_RSLICE,), jnp.float32),
            pltpu.VMEM((_K,), jnp.int32),
            pltpu.VMEM((_K,), jnp.int32),
            pltpu.VMEM((_K,), jnp.float32),
            pltpu.VMEM_SHARED((_NTS * _NPAD,), jnp.float32),
            pltpu.VMEM_SHARED((_NPAD,), jnp.float32),
            pltpu.SemaphoreType.DMA,
        ],
        compiler_params=pltpu.CompilerParams(needs_layout_passes=False),
    )
    def alpha_kernel(sT_hbm, src_hbm, dst_hbm, al_hbm,
                     s_src, s_dst, dloc, tmp, srci, dsti, evb,
                     dparts, dfin, sem):
        cid = lax.axis_index("c")
        sid = lax.axis_index("s")
        rs = sid * _RSLICE

        def per_head(head, carry):
            pltpu.sync_copy(sT_hbm.at[pl.ds(2 * head * _NPAD, _NPAD)], s_src)
            pltpu.sync_copy(sT_hbm.at[pl.ds((2 * head + 1) * _NPAD, _NPAD)],
                            s_dst)

            def zden(i, c):
                dloc[pl.ds(i * 16, 16)] = jnp.zeros((16,), jnp.float32)
                return c
            lax.fori_loop(0, _NPAD // 16, zden, 0)

            def block_a(b, c):
                off = (sid * nblk_den + b) * _K
                pltpu.sync_copy(src_hbm.at[pl.ds(off, _K)], srci)
                pltpu.sync_copy(dst_hbm.at[pl.ds(off, _K)], dsti)
                for i in range(_K // 16):
                    sv = srci[pl.ds(i * 16, 16)]
                    dv = dsti[pl.ds(i * 16, 16)]
                    z = plsc.load_gather(s_src, [sv]) + plsc.load_gather(s_dst, [dv])
                    plsc.addupdate_scatter(dloc, [dv], _lrelu_exp(z))
                return c
            lax.fori_loop(0, nblk_den, block_a, 0)

            pltpu.sync_copy(dloc, dparts.at[pl.ds(sid * _NPAD, _NPAD)])
            plsc.subcore_barrier()

            def zt(i, c):
                tmp[pl.ds(i * 16, 16)] = jnp.zeros((16,), jnp.float32)
                return c
            lax.fori_loop(0, _RSLICE // 16, zt, 0)

            def red(t, c):
                pltpu.sync_copy(dparts.at[pl.ds(t * _NPAD + rs, _RSLICE)],
                                dloc.at[pl.ds(0, _RSLICE)])

                def addt(i, c2):
                    tmp[pl.ds(i * 16, 16)] = (tmp[pl.ds(i * 16, 16)]
                                              + dloc[pl.ds(i * 16, 16)])
                    return c2
                lax.fori_loop(0, _RSLICE // 16, addt, 0)
                return c
            lax.fori_loop(0, _NTS, red, 0)

            def rec(i, c):
                tmp[pl.ds(i * 16, 16)] = 1.0 / tmp[pl.ds(i * 16, 16)]
                return c
            lax.fori_loop(0, _RSLICE // 16, rec, 0)
            pltpu.sync_copy(tmp, dfin.at[pl.ds(rs, _RSLICE)])
            plsc.subcore_barrier()
            pltpu.sync_copy(dfin, dloc)

            def block_b(b, c):
                off = ((cid * _NTS + sid) * nblk_al + b) * _K
                pltpu.sync_copy(src_hbm.at[pl.ds(off, _K)], srci)
                pltpu.sync_copy(dst_hbm.at[pl.ds(off, _K)], dsti)
                for i in range(_K // 16):
                    sv = srci[pl.ds(i * 16, 16)]
                    dv = dsti[pl.ds(i * 16, 16)]
                    z = plsc.load_gather(s_src, [sv]) + plsc.load_gather(s_dst, [dv])
                    evb[pl.ds(i * 16, 16)] = (_lrelu_exp(z)
                                              * plsc.load_gather(dloc, [dv]))
                pltpu.sync_copy(evb, al_hbm.at[pl.ds(head * epad + off, _K)])
                return c
            lax.fori_loop(0, nblk_al, block_b, 0)
            plsc.subcore_barrier()
            return carry

        lax.fori_loop(0, 3, per_head, 0)

    return alpha_kernel


def _make_agg_kernel(epad):
    nblocks = epad // (_NTS * _K)
    ngroups = nblocks // _GB

    @functools.partial(
        pl.kernel,
        out_type=jax.ShapeDtypeStruct((_NPAD, 3 * _H), jnp.float32),
        mesh=_MESH,
        scratch_types=[
            pltpu.VMEM((_GB, _K), jnp.int32),
            pltpu.VMEM((_GB, _K), jnp.int32),
            pltpu.VMEM((_GB * _K,), jnp.float32),
            pltpu.VMEM((2 * _K, 128), jnp.float32),
            pltpu.VMEM((2 * _CB, 128), jnp.float32),
            pltpu.VMEM((128,), jnp.float32),
            pltpu.VMEM((128,), jnp.float32),
            pltpu.VMEM_SHARED((_NPAD, 128), jnp.float32),
            pltpu.SemaphoreType.DMA,
            pltpu.SemaphoreType.DMA,
        ],
        compiler_params=pltpu.CompilerParams(needs_layout_passes=False),
    )
    def agg_kernel(h_hbm, al_hbm, src_hbm, dst_hbm, p0_hbm, fin_hbm, out_hbm,
                   srci, dsti, ab, rows, cbuf, p0v, finv, acc,
                   semg, semc):
        cid = lax.axis_index("c")
        sid = lax.axis_index("s")
        rs = sid * _RSLICE

        def per_head(head, carry):
            def zcb(j, c):
                for cc in range(8):
                    cbuf[j, pl.ds(cc * 16, 16)] = jnp.zeros((16,), jnp.float32)
                return c
            lax.fori_loop(0, _CB, zcb, 0)

            def zacc(chunk, c):
                pltpu.sync_copy(cbuf.at[pl.ds(0, _CB)],
                                acc.at[pl.ds(rs + chunk * _CB, _CB)])
                return c
            lax.fori_loop(0, _RSLICE // _CB, zacc, 0)
            plsc.subcore_barrier()

            def group(g, c):
                brow = sid * nblocks + g * _GB
                pltpu.sync_copy(src_hbm.at[pl.ds(brow, _GB)], srci)
                pltpu.sync_copy(dst_hbm.at[pl.ds(brow, _GB)], dsti)
                pltpu.sync_copy(al_hbm.at[pl.ds(head * epad + brow * _K,
                                                _GB * _K)], ab)
                pltpu.async_copy(h_hbm.at[head, cid].at[srci.at[0]],
                                 rows.at[pl.ds(0, _K)], semg)

                def block(b2, c2):
                    par = lax.rem(b2, 2)

                    @pl.when(b2 + 1 < _GB)
                    def _():
                        pltpu.async_copy(
                            h_hbm.at[head, cid].at[srci.at[b2 + 1]],
                            rows.at[pl.ds(lax.rem(b2 + 1, 2) * _K, _K)], semg)

                    pltpu.make_async_copy(h_hbm.at[head, cid].at[pl.ds(0, _K)],
                                          rows.at[pl.ds(par * _K, _K)],
                                          semg).wait()

                    def scale(j, c3):
                        av = plsc.load_gather(ab, [_bcast16(b2 * _K + j)])
                        rw = rows.at[pl.ds(par * _K, _K)]
                        for cc in range(8):
                            rw[j, pl.ds(cc * 16, 16)] = (
                                rw[j, pl.ds(cc * 16, 16)] * av)
                        return c3
                    lax.fori_loop(0, _K, scale, 0)
                    pltpu.sync_copy(rows.at[pl.ds(par * _K, _K)],
                                    acc.at[dsti.at[b2]], add=True)
                    return c2
                lax.fori_loop(0, _GB, block, 0)
                return c
            lax.fori_loop(0, ngroups, group, 0)
            plsc.subcore_barrier()

            pltpu.sync_copy(p0_hbm.at[pl.ds(cid * 128, 128)], p0v)
            pltpu.sync_copy(fin_hbm.at[pl.ds(head * 256 + cid * 128, 128)], finv)
            col0 = head * _H + cid * 128
            nchunks = _RSLICE // _CB

            def act1(v):
                return v / (1.0 + jnp.exp(-v))

            def act2(v):
                return 1.0 - 2.0 / (1.0 + jnp.exp(2.0 * v))

            def mk_drain(actfn, use_p0):
                def drain():
                    pltpu.async_copy(acc.at[pl.ds(rs, _CB)],
                                     cbuf.at[pl.ds(0, _CB)], semc)

                    def chunkf(chunk, c):
                        par = lax.rem(chunk, 2)

                        @pl.when(chunk + 1 < nchunks)
                        def _():
                            pltpu.async_copy(
                                acc.at[pl.ds(rs + (chunk + 1) * _CB, _CB)],
                                cbuf.at[pl.ds(lax.rem(chunk + 1, 2) * _CB, _CB)],
                                semc)
                        pltpu.make_async_copy(acc.at[pl.ds(rs, _CB)],
                                              cbuf.at[pl.ds(par * _CB, _CB)],
                                              semc).wait()

                        def rowact(j, c2):
                            cb = cbuf.at[pl.ds(par * _CB, _CB)]
                            for cc in range(8):
                                v = cb[j, pl.ds(cc * 16, 16)]
                                if use_p0:
                                    pa = p0v[pl.ds(cc * 16, 16)]
                                    v = jnp.where(v >= 0, v, pa * v)
                                else:
                                    v = actfn(v)
                                fa = finv[pl.ds(cc * 16, 16)]
                                cb[j, pl.ds(cc * 16, 16)] = jnp.where(
                                    v >= 0, v, fa * v)
                            return c2
                        lax.fori_loop(0, _CB, rowact, 0)
                        pltpu.sync_copy(cbuf.at[pl.ds(par * _CB, _CB)],
                                        out_hbm.at[pl.ds(rs + chunk * _CB, _CB),
                                                   pl.ds(col0, 128)])
                        return c
                    lax.fori_loop(0, nchunks, chunkf, 0)
                    return 0
                return drain

            lax.switch(head, [mk_drain(None, True), mk_drain(act1, False),
                              mk_drain(act2, False)])
            plsc.subcore_barrier()
            return carry

        lax.fori_loop(0, 3, per_head, 0)

    return agg_kernel


def kernel(node_features, edge_index, W0, a_src0, a_dst0, prelu0_alpha,
           W1, a_src1, a_dst1, W2, a_src2, a_dst2, final_prelu_alpha):
    n, d = node_features.shape
    e = edge_index.shape[1]
    etot = e + n
    ealign = _NTS * _K * _GB
    epad = -(-etot // ealign) * ealign

    loops = jnp.arange(n, dtype=edge_index.dtype)
    src = jnp.concatenate([edge_index[0], loops,
                           jnp.zeros((epad - etot,), edge_index.dtype)])
    dst = jnp.concatenate([edge_index[1], loops,
                           jnp.full((epad - etot,), n, edge_index.dtype)])
    src2 = src.reshape(epad // _K, _K)
    dst2 = dst.reshape(epad // _K, _K)

    x_pad = jnp.pad(node_features, ((0, _NPAD - n), (0, 0)))
    w_all = jnp.stack([W0, W1, W2]).reshape(3, d, 2, 128).transpose(0, 2, 1, 3)
    a_all = jnp.stack([a_src0, a_dst0, a_src1, a_dst1, a_src2, a_dst2])
    a_all = a_all.reshape(3, 2, 2, 128).transpose(0, 2, 1, 3)

    h_all, sT = _tc_proj(x_pad, w_all, a_all)
    sT = sT.reshape(6 * _NPAD)
    alpha = _make_alpha_kernel(epad)(sT, src, dst)
    out = _make_agg_kernel(epad)(h_all, alpha, src2, dst2,
                                 prelu0_alpha, final_prelu_alpha)
    return out[:n]


# speedup vs baseline: 1.0354x; 1.0354x over previous
import functools

import jax
import jax.numpy as jnp
from jax import lax
from jax.experimental import pallas as pl
from jax.experimental.pallas import tpu as pltpu
from jax.experimental.pallas import tpu_sc as plsc

_N = 10000
_D = 256
_H = 256
_NPAD = 10240
_R = 512
_NSC = 2
_NTS = 16
_K = 128
_RSLICE = _NPAD // _NTS
_CB = 32
_GB = 8


def _tc_proj(x_pad, w_all, a_all):
    def body(x_ref, w_ref, a_ref, h_ref, s_ref):
        cid = pl.program_id(2)
        xb = x_ref[...]
        hb = jnp.dot(xb, w_ref[0, 0], preferred_element_type=jnp.float32)
        h_ref[0, 0] = hb
        sb = lax.dot_general(a_ref[0, 0], hb, (((1,), (1,)), ((), ())),
                             preferred_element_type=jnp.float32)

        @pl.when(cid == 0)
        def _():
            s_ref[0] = sb

        @pl.when(cid == 1)
        def _():
            s_ref[0] = s_ref[0] + sb

    nrb = _NPAD // _R
    return pl.pallas_call(
        body,
        grid=(3, nrb, 2),
        in_specs=[
            pl.BlockSpec((_R, _D), lambda h, r, c: (r, 0)),
            pl.BlockSpec((1, 1, _D, 128), lambda h, r, c: (h, c, 0, 0)),
            pl.BlockSpec((1, 1, 2, 128), lambda h, r, c: (h, c, 0, 0)),
        ],
        out_specs=[
            pl.BlockSpec((1, 1, _R, 128), lambda h, r, c: (h, c, r, 0)),
            pl.BlockSpec((1, 2, _R), lambda h, r, c: (h, 0, r)),
        ],
        out_shape=[
            jax.ShapeDtypeStruct((3, 2, _NPAD, 128), jnp.float32),
            jax.ShapeDtypeStruct((3, 2, _NPAD), jnp.float32),
        ],
    )(x_pad, w_all, a_all)


def _lrelu_exp(z):
    return jnp.exp(jnp.where(z >= 0, z, 0.2 * z))


def _bcast16(j):
    return jnp.zeros((16,), jnp.int32) + j


_MESH = plsc.VectorSubcoreMesh(core_axis_name="c", subcore_axis_name="s",
                               num_cores=_NSC, num_subcores=_NTS)


def _make_alpha_kernel(epad):
    nblk_den = epad // (_NTS * _K)
    nblk_al = epad // (_NSC * _NTS * _K)

    @functools.partial(
        pl.kernel,
        out_type=jax.ShapeDtypeStruct((3 * epad,), jnp.float32),
        mesh=_MESH,
        scratch_types=[
            pltpu.VMEM((_NPAD,), jnp.float32),
            pltpu.VMEM((_NPAD,), jnp.float32),
            pltpu.VMEM((_NPAD,), jnp.float32),
            pltpu.VMEM((_RSLICE,), jnp.float32),
            pltpu.VMEM((_K,), jnp.int32),
            pltpu.VMEM((_K,), jnp.int32),
            pltpu.VMEM((_K,), jnp.float32),
            pltpu.VMEM_SHARED((_NTS * _NPAD,), jnp.float32),
            pltpu.VMEM_SHARED((_NPAD,), jnp.float32),
            pltpu.SemaphoreType.DMA,
        ],
        compiler_params=pltpu.CompilerParams(needs_layout_passes=False),
    )
    def alpha_kernel(sT_hbm, src_hbm, dst_hbm, al_hbm,
                     s_src, s_dst, dloc, tmp, srci, dsti, evb,
                     dparts, dfin, sem):
        cid = lax.axis_index("c")
        sid = lax.axis_index("s")
        rs = sid * _RSLICE

        def per_head(head, carry):
            pltpu.sync_copy(sT_hbm.at[pl.ds(2 * head * _NPAD, _NPAD)], s_src)
            pltpu.sync_copy(sT_hbm.at[pl.ds((2 * head + 1) * _NPAD, _NPAD)],
                            s_dst)

            def zden(i, c):
                dloc[pl.ds(i * 16, 16)] = jnp.zeros((16,), jnp.float32)
                return c
            lax.fori_loop(0, _NPAD // 16, zden, 0)

            def block_a(b, c):
                off = (sid * nblk_den + b) * _K
                pltpu.sync_copy(src_hbm.at[pl.ds(off, _K)], srci)
                pltpu.sync_copy(dst_hbm.at[pl.ds(off, _K)], dsti)
                for i in range(_K // 16):
                    sv = srci[pl.ds(i * 16, 16)]
                    dv = dsti[pl.ds(i * 16, 16)]
                    z = plsc.load_gather(s_src, [sv]) + plsc.load_gather(s_dst, [dv])
                    plsc.addupdate_scatter(dloc, [dv], _lrelu_exp(z))
                return c
            lax.fori_loop(0, nblk_den, block_a, 0)

            pltpu.sync_copy(dloc, dparts.at[pl.ds(sid * _NPAD, _NPAD)])
            plsc.subcore_barrier()

            def zt(i, c):
                tmp[pl.ds(i * 16, 16)] = jnp.zeros((16,), jnp.float32)
                return c
            lax.fori_loop(0, _RSLICE // 16, zt, 0)

            def red(t, c):
                pltpu.sync_copy(dparts.at[pl.ds(t * _NPAD + rs, _RSLICE)],
                                dloc.at[pl.ds(0, _RSLICE)])

                def addt(i, c2):
                    tmp[pl.ds(i * 16, 16)] = (tmp[pl.ds(i * 16, 16)]
                                              + dloc[pl.ds(i * 16, 16)])
                    return c2
                lax.fori_loop(0, _RSLICE // 16, addt, 0)
                return c
            lax.fori_loop(0, _NTS, red, 0)

            def rec(i, c):
                tmp[pl.ds(i * 16, 16)] = 1.0 / tmp[pl.ds(i * 16, 16)]
                return c
            lax.fori_loop(0, _RSLICE // 16, rec, 0)
            pltpu.sync_copy(tmp, dfin.at[pl.ds(rs, _RSLICE)])
            plsc.subcore_barrier()
            pltpu.sync_copy(dfin, dloc)

            def block_b(b, c):
                off = ((cid * _NTS + sid) * nblk_al + b) * _K
                pltpu.sync_copy(src_hbm.at[pl.ds(off, _K)], srci)
                pltpu.sync_copy(dst_hbm.at[pl.ds(off, _K)], dsti)
                for i in range(_K // 16):
                    sv = srci[pl.ds(i * 16, 16)]
                    dv = dsti[pl.ds(i * 16, 16)]
                    z = plsc.load_gather(s_src, [sv]) + plsc.load_gather(s_dst, [dv])
                    evb[pl.ds(i * 16, 16)] = (_lrelu_exp(z)
                                              * plsc.load_gather(dloc, [dv]))
                pltpu.sync_copy(evb, al_hbm.at[pl.ds(head * epad + off, _K)])
                return c
            lax.fori_loop(0, nblk_al, block_b, 0)
            plsc.subcore_barrier()
            return carry

        lax.fori_loop(0, 3, per_head, 0)

    return alpha_kernel


def _make_agg_kernel(epad):
    nblocks = epad // (_NTS * _K)
    ngroups = nblocks // _GB

    @functools.partial(
        pl.kernel,
        out_type=jax.ShapeDtypeStruct((_NPAD, 3 * _H), jnp.float32),
        mesh=_MESH,
        scratch_types=[
            pltpu.VMEM((_GB, _K), jnp.int32),
            pltpu.VMEM((_GB, _K), jnp.int32),
            pltpu.VMEM((_GB * _K,), jnp.float32),
            pltpu.VMEM((_K, 128), jnp.float32),
            pltpu.VMEM((_K, 128), jnp.float32),
            pltpu.VMEM((_CB, 128), jnp.float32),
            pltpu.VMEM((_CB, 128), jnp.float32),
            pltpu.VMEM((128,), jnp.float32),
            pltpu.VMEM((128,), jnp.float32),
            pltpu.VMEM_SHARED((_NPAD, 128), jnp.float32),
            pltpu.SemaphoreType.DMA,
            pltpu.SemaphoreType.DMA,
            pltpu.SemaphoreType.DMA,
            pltpu.SemaphoreType.DMA,
        ],
        compiler_params=pltpu.CompilerParams(needs_layout_passes=False),
    )
    def agg_kernel(h_hbm, al_hbm, src_hbm, dst_hbm, p0_hbm, fin_hbm, out_hbm,
                   srci, dsti, ab, rowsA, rowsB, cbufA, cbufB, p0v, finv, acc,
                   semA, semB, semcA, semcB):
        cid = lax.axis_index("c")
        sid = lax.axis_index("s")
        rs = sid * _RSLICE
        rbufs = (rowsA, rowsB)
        rsems = (semA, semB)
        cbufs = (cbufA, cbufB)
        csems = (semcA, semcB)

        def per_head(head, carry):
            def zcb(j, c):
                for cc in range(8):
                    cbufA[j, pl.ds(cc * 16, 16)] = jnp.zeros((16,), jnp.float32)
                return c
            lax.fori_loop(0, _CB, zcb, 0)

            def zacc(chunk, c):
                pltpu.sync_copy(cbufA, acc.at[pl.ds(rs + chunk * _CB, _CB)])
                return c
            lax.fori_loop(0, _RSLICE // _CB, zacc, 0)
            plsc.subcore_barrier()

            def do_block(blk):
                buf = rbufs[blk % 2]
                sem = rsems[blk % 2]
                pltpu.make_async_copy(h_hbm.at[head, cid].at[pl.ds(0, _K)],
                                      buf, sem).wait()

                def scale(j, c3):
                    av = plsc.load_gather(ab, [_bcast16(blk * _K + j)])
                    for cc in range(8):
                        buf[j, pl.ds(cc * 16, 16)] = (
                            buf[j, pl.ds(cc * 16, 16)] * av)
                    return c3
                lax.fori_loop(0, _K, scale, 0)
                pltpu.sync_copy(buf, acc.at[dsti.at[blk]], add=True)

            def fire(blk):
                pltpu.async_copy(h_hbm.at[head, cid].at[srci.at[blk]],
                                 rbufs[blk % 2], rsems[blk % 2])

            def group(g, c):
                brow = sid * nblocks + g * _GB
                pltpu.sync_copy(src_hbm.at[pl.ds(brow, _GB)], srci)
                pltpu.sync_copy(dst_hbm.at[pl.ds(brow, _GB)], dsti)
                pltpu.sync_copy(al_hbm.at[pl.ds(head * epad + brow * _K,
                                                _GB * _K)], ab)
                fire(0)
                fire(1)
                for blk in range(_GB):
                    do_block(blk)
                    if blk + 2 < _GB:
                        fire(blk + 2)
                return c
            lax.fori_loop(0, ngroups, group, 0)
            plsc.subcore_barrier()

            pltpu.sync_copy(p0_hbm.at[pl.ds(cid * 128, 128)], p0v)
            pltpu.sync_copy(fin_hbm.at[pl.ds(head * 256 + cid * 128, 128)], finv)
            col0 = head * _H + cid * 128
            npairs = _RSLICE // _CB // 2

            def act1(v):
                return v / (1.0 + jnp.exp(-v))

            def act2(v):
                return 1.0 - 2.0 / (1.0 + jnp.exp(2.0 * v))

            def mk_drain(actfn, use_p0):
                def half(cp, side):
                    chunk = 2 * cp + side
                    cb = cbufs[side]
                    pltpu.make_async_copy(acc.at[pl.ds(rs, _CB)], cb,
                                          csems[side]).wait()

                    def rowact(j, c2):
                        for cc in range(8):
                            v = cb[j, pl.ds(cc * 16, 16)]
                            if use_p0:
                                pa = p0v[pl.ds(cc * 16, 16)]
                                v = jnp.where(v >= 0, v, pa * v)
                            else:
                                v = actfn(v)
                            fa = finv[pl.ds(cc * 16, 16)]
                            cb[j, pl.ds(cc * 16, 16)] = jnp.where(
                                v >= 0, v, fa * v)
                        return c2
                    lax.fori_loop(0, _CB, rowact, 0)
                    pltpu.sync_copy(cb,
                                    out_hbm.at[pl.ds(rs + chunk * _CB, _CB),
                                               pl.ds(col0, 128)])

                def stage(chunk, side):
                    pltpu.async_copy(acc.at[pl.ds(rs + chunk * _CB, _CB)],
                                     cbufs[side], csems[side])

                def drain():
                    stage(0, 0)

                    def pair(cp, c):
                        stage(2 * cp + 1, 1)
                        half(cp, 0)

                        @pl.when(cp + 1 < npairs)
                        def _():
                            stage(2 * cp + 2, 0)
                        half(cp, 1)
                        return c
                    lax.fori_loop(0, npairs, pair, 0)
                    return 0
                return drain

            lax.switch(head, [mk_drain(None, True), mk_drain(act1, False),
                              mk_drain(act2, False)])
            plsc.subcore_barrier()
            return carry

        lax.fori_loop(0, 3, per_head, 0)

    return agg_kernel


def kernel(node_features, edge_index, W0, a_src0, a_dst0, prelu0_alpha,
           W1, a_src1, a_dst1, W2, a_src2, a_dst2, final_prelu_alpha):
    n, d = node_features.shape
    e = edge_index.shape[1]
    etot = e + n
    ealign = _NTS * _K * _GB
    epad = -(-etot // ealign) * ealign

    loops = jnp.arange(n, dtype=edge_index.dtype)
    src = jnp.concatenate([edge_index[0], loops,
                           jnp.zeros((epad - etot,), edge_index.dtype)])
    dst = jnp.concatenate([edge_index[1], loops,
                           jnp.full((epad - etot,), n, edge_index.dtype)])
    src2 = src.reshape(epad // _K, _K)
    dst2 = dst.reshape(epad // _K, _K)

    x_pad = jnp.pad(node_features, ((0, _NPAD - n), (0, 0)))
    w_all = jnp.stack([W0, W1, W2]).reshape(3, d, 2, 128).transpose(0, 2, 1, 3)
    a_all = jnp.stack([a_src0, a_dst0, a_src1, a_dst1, a_src2, a_dst2])
    a_all = a_all.reshape(3, 2, 2, 128).transpose(0, 2, 1, 3)

    h_all, sT = _tc_proj(x_pad, w_all, a_all)
    sT = sT.reshape(6 * _NPAD)
    alpha = _make_alpha_kernel(epad)(sT, src, dst)
    out = _make_agg_kernel(epad)(h_all, alpha, src2, dst2,
                                 prelu0_alpha, final_prelu_alpha)
    return out[:n]


# speedup vs baseline: 1.4825x; 1.4318x over previous
import functools

import jax
import jax.numpy as jnp
from jax import lax
from jax.experimental import pallas as pl
from jax.experimental.pallas import tpu as pltpu
from jax.experimental.pallas import tpu_sc as plsc

_N = 10000
_D = 256
_H = 256
_NPAD = 10240
_R = 512
_NSC = 2
_NTS = 16
_K = 128
_RSLICE = _NPAD // _NTS
_CB = 32


def _tc_proj(x_pad, w_all, a_all):
    def body(x_ref, w_ref, a_ref, h_ref, s_ref):
        cid = pl.program_id(2)
        xb = x_ref[...]
        hb = jnp.dot(xb, w_ref[0, 0], preferred_element_type=jnp.float32)
        h_ref[0, 0] = hb
        sb = lax.dot_general(a_ref[0, 0], hb, (((1,), (1,)), ((), ())),
                             preferred_element_type=jnp.float32)

        @pl.when(cid == 0)
        def _():
            s_ref[0] = sb

        @pl.when(cid == 1)
        def _():
            s_ref[0] = s_ref[0] + sb

    nrb = _NPAD // _R
    return pl.pallas_call(
        body,
        grid=(3, nrb, 2),
        in_specs=[
            pl.BlockSpec((_R, _D), lambda h, r, c: (r, 0)),
            pl.BlockSpec((1, 1, _D, 128), lambda h, r, c: (h, c, 0, 0)),
            pl.BlockSpec((1, 1, 2, 128), lambda h, r, c: (h, c, 0, 0)),
        ],
        out_specs=[
            pl.BlockSpec((1, 1, _R, 128), lambda h, r, c: (h, c, r, 0)),
            pl.BlockSpec((1, 2, _R), lambda h, r, c: (h, 0, r)),
        ],
        out_shape=[
            jax.ShapeDtypeStruct((3, 2, _NPAD, 128), jnp.float32),
            jax.ShapeDtypeStruct((3, 2, _NPAD), jnp.float32),
        ],
    )(x_pad, w_all, a_all)


def _lrelu_exp(z):
    return jnp.exp(jnp.where(z >= 0, z, 0.2 * z))


def _bcast16(j):
    return jnp.zeros((16,), jnp.int32) + j


_MESH = plsc.VectorSubcoreMesh(core_axis_name="c", subcore_axis_name="s",
                               num_cores=_NSC, num_subcores=_NTS)


def _make_alpha_kernel(epad):
    nblk_den = epad // (_NTS * _K)
    nblk_al = epad // (_NSC * _NTS * _K)

    @functools.partial(
        pl.kernel,
        out_type=jax.ShapeDtypeStruct((3 * epad,), jnp.float32),
        mesh=_MESH,
        scratch_types=[
            pltpu.VMEM((_NPAD,), jnp.float32),
            pltpu.VMEM((_NPAD,), jnp.float32),
            pltpu.VMEM((_NPAD,), jnp.float32),
            pltpu.VMEM((_RSLICE,), jnp.float32),
            pltpu.VMEM((2, _K), jnp.int32),
            pltpu.VMEM((_K,), jnp.float32),
            pltpu.VMEM_SHARED((_NTS * _NPAD,), jnp.float32),
            pltpu.VMEM_SHARED((_NPAD,), jnp.float32),
            pltpu.SemaphoreType.DMA,
        ],
        compiler_params=pltpu.CompilerParams(needs_layout_passes=False),
    )
    def alpha_kernel(sT_hbm, sd_hbm, al_hbm,
                     s_src, s_dst, dloc, tmp, sdi, evb,
                     dparts, dfin, sem):
        cid = lax.axis_index("c")
        sid = lax.axis_index("s")
        rs = sid * _RSLICE

        def per_head(head, carry):
            pltpu.sync_copy(sT_hbm.at[pl.ds(2 * head * _NPAD, _NPAD)], s_src)
            pltpu.sync_copy(sT_hbm.at[pl.ds((2 * head + 1) * _NPAD, _NPAD)],
                            s_dst)

            def zden(i, c):
                dloc[pl.ds(i * 16, 16)] = jnp.zeros((16,), jnp.float32)
                return c
            lax.fori_loop(0, _NPAD // 16, zden, 0)

            def block_a(b, c):
                pltpu.sync_copy(sd_hbm.at[sid * nblk_den + b], sdi)
                for i in range(_K // 16):
                    sv = sdi[0, pl.ds(i * 16, 16)]
                    dv = sdi[1, pl.ds(i * 16, 16)]
                    z = plsc.load_gather(s_src, [sv]) + plsc.load_gather(s_dst, [dv])
                    plsc.addupdate_scatter(dloc, [dv], _lrelu_exp(z))
                return c
            lax.fori_loop(0, nblk_den, block_a, 0)

            pltpu.sync_copy(dloc, dparts.at[pl.ds(sid * _NPAD, _NPAD)])
            plsc.subcore_barrier()

            def zt(i, c):
                tmp[pl.ds(i * 16, 16)] = jnp.zeros((16,), jnp.float32)
                return c
            lax.fori_loop(0, _RSLICE // 16, zt, 0)

            def red(t, c):
                pltpu.sync_copy(dparts.at[pl.ds(t * _NPAD + rs, _RSLICE)],
                                dloc.at[pl.ds(0, _RSLICE)])

                def addt(i, c2):
                    tmp[pl.ds(i * 16, 16)] = (tmp[pl.ds(i * 16, 16)]
                                              + dloc[pl.ds(i * 16, 16)])
                    return c2
                lax.fori_loop(0, _RSLICE // 16, addt, 0)
                return c
            lax.fori_loop(0, _NTS, red, 0)

            def rec(i, c):
                tmp[pl.ds(i * 16, 16)] = 1.0 / tmp[pl.ds(i * 16, 16)]
                return c
            lax.fori_loop(0, _RSLICE // 16, rec, 0)
            pltpu.sync_copy(tmp, dfin.at[pl.ds(rs, _RSLICE)])
            plsc.subcore_barrier()
            pltpu.sync_copy(dfin, dloc)

            def block_b(b, c):
                blk = (cid * _NTS + sid) * nblk_al + b
                pltpu.sync_copy(sd_hbm.at[blk], sdi)
                for i in range(_K // 16):
                    sv = sdi[0, pl.ds(i * 16, 16)]
                    dv = sdi[1, pl.ds(i * 16, 16)]
                    z = plsc.load_gather(s_src, [sv]) + plsc.load_gather(s_dst, [dv])
                    evb[pl.ds(i * 16, 16)] = (_lrelu_exp(z)
                                              * plsc.load_gather(dloc, [dv]))
                pltpu.sync_copy(evb, al_hbm.at[pl.ds(head * epad + blk * _K, _K)])
                return c
            lax.fori_loop(0, nblk_al, block_b, 0)
            plsc.subcore_barrier()
            return carry

        lax.fori_loop(0, 3, per_head, 0)

    return alpha_kernel


def _make_agg_kernel(epad):
    nblocks = epad // (_NTS * _K)

    @functools.partial(
        pl.kernel,
        out_type=jax.ShapeDtypeStruct((_NPAD, 3 * _H), jnp.float32),
        mesh=_MESH,
        scratch_types=[
            pltpu.VMEM((2, _K), jnp.int32),
            pltpu.VMEM((_K,), jnp.float32),
            pltpu.VMEM((_K, 128), jnp.float32),
            pltpu.VMEM((_CB, 128), jnp.float32),
            pltpu.VMEM((128,), jnp.float32),
            pltpu.VMEM((128,), jnp.float32),
            pltpu.VMEM_SHARED((_NPAD, 128), jnp.float32),
            pltpu.SemaphoreType.DMA,
        ],
        compiler_params=pltpu.CompilerParams(needs_layout_passes=False),
    )
    def agg_kernel(h_hbm, al_hbm, sd_hbm, p0_hbm, fin_hbm, out_hbm,
                   sdi, ab, rows, cbuf, p0v, finv, acc, sem):
        cid = lax.axis_index("c")
        sid = lax.axis_index("s")
        rs = sid * _RSLICE

        def per_head(head, carry):
            def zcb(j, c):
                for cc in range(8):
                    cbuf[j, pl.ds(cc * 16, 16)] = jnp.zeros((16,), jnp.float32)
                return c
            lax.fori_loop(0, _CB, zcb, 0)

            def zacc(chunk, c):
                pltpu.sync_copy(cbuf, acc.at[pl.ds(rs + chunk * _CB, _CB)])
                return c
            lax.fori_loop(0, _RSLICE // _CB, zacc, 0)
            plsc.subcore_barrier()

            def block_b(b, c):
                blk = sid * nblocks + b
                pltpu.sync_copy(sd_hbm.at[blk], sdi)
                cp = pltpu.async_copy(h_hbm.at[head, cid].at[sdi.at[0]],
                                      rows, sem)
                pltpu.sync_copy(al_hbm.at[pl.ds(head * epad + blk * _K, _K)],
                                ab)
                cp.wait()

                def scale(j, c2):
                    av = plsc.load_gather(ab, [_bcast16(j)])
                    for cc in range(8):
                        rows[j, pl.ds(cc * 16, 16)] = (
                            rows[j, pl.ds(cc * 16, 16)] * av)
                    return c2
                lax.fori_loop(0, _K, scale, 0)
                pltpu.sync_copy(rows, acc.at[sdi.at[1]], add=True)
                return c
            lax.fori_loop(0, nblocks, block_b, 0)
            plsc.subcore_barrier()

            pltpu.sync_copy(p0_hbm.at[pl.ds(cid * 128, 128)], p0v)
            pltpu.sync_copy(fin_hbm.at[pl.ds(head * 256 + cid * 128, 128)], finv)
            col0 = head * _H + cid * 128

            def outchunk(chunk, c):
                r0 = rs + chunk * _CB
                pltpu.sync_copy(acc.at[pl.ds(r0, _CB)], cbuf)

                def rowact(j, c2):
                    for cc in range(8):
                        v = cbuf[j, pl.ds(cc * 16, 16)]
                        pa = p0v[pl.ds(cc * 16, 16)]
                        v0 = jnp.where(v >= 0, v, pa * v)
                        v1 = v / (1.0 + jnp.exp(-v))
                        v2 = 1.0 - 2.0 / (1.0 + jnp.exp(2.0 * v))
                        v = jnp.where(head == 0, v0,
                                      jnp.where(head == 1, v1, v2))
                        fa = finv[pl.ds(cc * 16, 16)]
                        v = jnp.where(v >= 0, v, fa * v)
                        cbuf[j, pl.ds(cc * 16, 16)] = v
                    return c2
                lax.fori_loop(0, _CB, rowact, 0)
                pltpu.sync_copy(cbuf,
                                out_hbm.at[pl.ds(r0, _CB), pl.ds(col0, 128)])
                return c
            lax.fori_loop(0, _RSLICE // _CB, outchunk, 0)
            plsc.subcore_barrier()
            return carry

        lax.fori_loop(0, 3, per_head, 0)

    return agg_kernel


def kernel(node_features, edge_index, W0, a_src0, a_dst0, prelu0_alpha,
           W1, a_src1, a_dst1, W2, a_src2, a_dst2, final_prelu_alpha):
    n, d = node_features.shape
    e = edge_index.shape[1]
    etot = e + n
    ealign = _NSC * _NTS * _K
    epad = -(-etot // ealign) * ealign

    loops = jnp.arange(n, dtype=edge_index.dtype)
    src = jnp.concatenate([edge_index[0], loops,
                           jnp.zeros((epad - etot,), edge_index.dtype)])
    dst = jnp.concatenate([edge_index[1], loops,
                           jnp.full((epad - etot,), n, edge_index.dtype)])
    sd = jnp.stack([src.reshape(epad // _K, _K),
                    dst.reshape(epad // _K, _K)], axis=1)

    x_pad = jnp.pad(node_features, ((0, _NPAD - n), (0, 0)))
    w_all = jnp.stack([W0, W1, W2]).reshape(3, d, 2, 128).transpose(0, 2, 1, 3)
    a_all = jnp.stack([a_src0, a_dst0, a_src1, a_dst1, a_src2, a_dst2])
    a_all = a_all.reshape(3, 2, 2, 128).transpose(0, 2, 1, 3)

    h_all, sT = _tc_proj(x_pad, w_all, a_all)
    sT = sT.reshape(6 * _NPAD)
    alpha = _make_alpha_kernel(epad)(sT, sd)
    out = _make_agg_kernel(epad)(h_all, alpha, sd,
                                 prelu0_alpha, final_prelu_alpha)
    return out[:n]
